# baseline (speedup 1.0000x reference)
import numpy as np
import jax
import jax.numpy as jnp
from jax.experimental import pallas as pl
from jax.experimental.pallas import tpu as pltpu

_BIN_SIZE = 12
_BIN_INTER = 2.0 * np.pi / _BIN_SIZE
_BN_INV = 1.0 / np.sqrt(1.0 + 1e-5)
_GRID = 2


def _colv(v):
    return jnp.transpose(v.reshape(1, -1))


def _head_decode_kernel(
    pts_ref,
    feats_ref,
    bw1_ref, bb1_ref, bg_ref, bbe_ref, bw2_ref, bb2_ref,
    cw1_ref, cb1_ref, cg_ref, cbe_ref, cw2_ref, cb2_ref,
    ms_ref,
    out_ref,
    w1_s,
    w2_s,
    sb_s,
    b2_s,
):
    G, C, K = feats_ref.shape
    N = G * K
    D = bw2_ref.shape[0]
    D2 = cw2_ref.shape[0]

    @pl.when(pl.program_id(0) == 0)
    def _prep():
        w1_s[:] = jnp.concatenate([bw1_ref[:], cw1_ref[:]], axis=0)
        zb = jnp.zeros_like(bw2_ref[:])
        zc = jnp.zeros_like(cw2_ref[:])
        w2_s[:D] = jnp.concatenate([bw2_ref[:], zb], axis=1)
        w2_s[D:D + D2] = jnp.concatenate([zc, cw2_ref[:]], axis=1)
        sc = _colv(jnp.concatenate([bg_ref[:], cg_ref[:]])) * _BN_INV
        b1c = _colv(jnp.concatenate([bb1_ref[:], cb1_ref[:]]))
        bec = _colv(jnp.concatenate([bbe_ref[:], cbe_ref[:]]))
        sb_s[:, 0:1] = sc
        sb_s[:, 1:2] = b1c * sc + bec
        b2_s[:D] = _colv(bb2_ref[:])
        b2_s[D:D + D2] = _colv(cb2_ref[:])

    feats = jnp.concatenate([feats_ref[b] for b in range(G)], axis=1)
    h = jnp.dot(w1_s[:], feats, preferred_element_type=jnp.float32)
    h = jnp.maximum(h * sb_s[:, 0:1] + sb_s[:, 1:2], 0.0)

    enc = jnp.dot(w2_s[:], h, preferred_element_type=jnp.float32) + b2_s[:]
    box_enc = enc[:D]
    clsv = enc[D:D + D2]

    c0, c1, c2 = clsv[0:1], clsv[1:2], clsv[2:3]
    cls_idx = jnp.where(c1 > c0, 1, 0)
    cls_idx = jnp.where(c2 > jnp.maximum(c0, c1), 2, cls_idx)

    def _anchor(col):
        return jnp.where(
            cls_idx == 0, ms_ref[0, col],
            jnp.where(cls_idx == 1, ms_ref[1, col], ms_ref[2, col]))

    dxa, dya, dza = _anchor(0), _anchor(1), _anchor(2)
    diagonal = jnp.sqrt(dxa * dxa + dya * dya)

    xyz = jnp.transpose(pts_ref[:].reshape(N, 3))
    xg = box_enc[0:1] * diagonal + xyz[0:1]
    yg = box_enc[1:2] * diagonal + xyz[1:2]
    zg = box_enc[2:3] * dza + xyz[2:3]
    dxg = jnp.exp(box_enc[3:4]) * dxa
    dyg = jnp.exp(box_enc[4:5]) * dya
    dzg = jnp.exp(box_enc[5:6]) * dza

    best = box_enc[6:7]
    bid = jnp.zeros_like(best, dtype=jnp.int32)
    res = box_enc[18:19]
    for i in range(1, _BIN_SIZE):
        cur = box_enc[6 + i:7 + i]
        gt = cur > best
        bid = jnp.where(gt, i, bid)
        res = jnp.where(gt, box_enc[18 + i:19 + i], res)
        best = jnp.maximum(best, cur)
    rg = (bid.astype(jnp.float32) * _BIN_INTER - np.pi + _BIN_INTER / 2.0
          + res * (_BIN_INTER / 2.0))

    rows = jnp.concatenate([xg, yg, zg, dxg, dyg, dzg, rg], axis=0)
    out_ref[:] = jnp.transpose(rows).reshape(G, K, 7)


def kernel(ctr_preds, ctr_feats, gt_boxes, points, gt_labels, box_w1, box_b1,
           box_gamma, box_beta, box_w2, box_b2, cls_w1, cls_b1, cls_gamma,
           cls_beta, cls_w2, cls_b2, mean_size):
    B, C, K = ctr_feats.shape
    G = B // _GRID
    M2 = box_w1.shape[0] + cls_w1.shape[0]
    D2 = box_w2.shape[0] + cls_w2.shape[0]

    full = pl.BlockSpec(index_map=lambda g: tuple([0]))
    full2 = pl.BlockSpec(index_map=lambda g: (0, 0))
    group3 = lambda d2, d3: pl.BlockSpec((G, d2, d3), lambda g: (g, 0, 0))

    return pl.pallas_call(
        _head_decode_kernel,
        grid=(_GRID,),
        in_specs=[
            group3(K, 3),
            group3(C, K),
            full2, full, full, full, full2, full,
            full2, full, full, full, full2, full,
            pl.BlockSpec(memory_space=pltpu.SMEM),
        ],
        out_specs=group3(K, 7),
        out_shape=jax.ShapeDtypeStruct((B, K, 7), jnp.float32),
        scratch_shapes=[
            pltpu.VMEM((M2, C), jnp.float32),
            pltpu.VMEM((D2, M2), jnp.float32),
            pltpu.VMEM((M2, 2), jnp.float32),
            pltpu.VMEM((D2, 1), jnp.float32),
        ],
    )(ctr_preds, ctr_feats,
      box_w1, box_b1, box_gamma, box_beta, box_w2, box_b2,
      cls_w1, cls_b1, cls_gamma, cls_beta, cls_w2, cls_b2,
      mean_size)


# speedup vs baseline: 1.0162x; 1.0162x over previous
import numpy as np
import jax
import jax.numpy as jnp
from jax.experimental import pallas as pl
from jax.experimental.pallas import tpu as pltpu

_BIN_SIZE = 12
_BIN_INTER = 2.0 * np.pi / _BIN_SIZE
_BN_INV = 1.0 / np.sqrt(1.0 + 1e-5)
_GRID = 2


def _colv(v):
    return jnp.transpose(v.reshape(1, -1))


def _head_decode_kernel(
    pts_ref,
    feats_ref,
    bw1_ref, bb1_ref, bg_ref, bbe_ref, bw2_ref, bb2_ref,
    cw1_ref, cb1_ref, cg_ref, cbe_ref, cw2_ref, cb2_ref,
    ms_ref,
    out_ref,
):
    G, C, K = feats_ref.shape
    N = G * K
    feats = jnp.concatenate([feats_ref[b] for b in range(G)], axis=1)

    w1 = jnp.concatenate([bw1_ref[:], cw1_ref[:]], axis=0)
    sc = _colv(jnp.concatenate([bg_ref[:], cg_ref[:]])) * _BN_INV
    b1c = _colv(jnp.concatenate([bb1_ref[:], cb1_ref[:]]))
    bec = _colv(jnp.concatenate([bbe_ref[:], cbe_ref[:]]))
    h = jnp.dot(w1, feats, preferred_element_type=jnp.float32)
    h = jnp.maximum(h * sc + (b1c * sc + bec), 0.0)

    M = h.shape[0] // 2
    D = bw2_ref.shape[0]
    zb = jnp.zeros_like(bw2_ref[:])
    zc = jnp.zeros_like(cw2_ref[:])
    w2 = jnp.concatenate([
        jnp.concatenate([bw2_ref[:], zb], axis=1),
        jnp.concatenate([zc, cw2_ref[:]], axis=1)], axis=0)
    b2c = _colv(jnp.concatenate([bb2_ref[:], cb2_ref[:]]))
    enc = jnp.dot(w2, h, preferred_element_type=jnp.float32) + b2c
    box_enc = enc[:D]
    clsv = enc[D:]

    c0, c1, c2 = clsv[0:1], clsv[1:2], clsv[2:3]
    cls_idx = jnp.where(c1 > c0, 1, 0)
    cls_idx = jnp.where(c2 > jnp.maximum(c0, c1), 2, cls_idx)

    def _anchor(col):
        return jnp.where(
            cls_idx == 0, ms_ref[0, col],
            jnp.where(cls_idx == 1, ms_ref[1, col], ms_ref[2, col]))

    dxa, dya, dza = _anchor(0), _anchor(1), _anchor(2)
    diagonal = jnp.sqrt(dxa * dxa + dya * dya)

    xyz = jnp.transpose(pts_ref[:].reshape(N, 3))
    xg = box_enc[0:1] * diagonal + xyz[0:1]
    yg = box_enc[1:2] * diagonal + xyz[1:2]
    zg = box_enc[2:3] * dza + xyz[2:3]
    dxg = jnp.exp(box_enc[3:4]) * dxa
    dyg = jnp.exp(box_enc[4:5]) * dya
    dzg = jnp.exp(box_enc[5:6]) * dza

    best = box_enc[6:7]
    bid = jnp.zeros_like(best, dtype=jnp.int32)
    res = box_enc[18:19]
    for i in range(1, _BIN_SIZE):
        cur = box_enc[6 + i:7 + i]
        gt = cur > best
        bid = jnp.where(gt, i, bid)
        res = jnp.where(gt, box_enc[18 + i:19 + i], res)
        best = jnp.maximum(best, cur)
    rg = (bid.astype(jnp.float32) * _BIN_INTER - np.pi + _BIN_INTER / 2.0
          + res * (_BIN_INTER / 2.0))

    rows = jnp.concatenate([xg, yg, zg, dxg, dyg, dzg, rg], axis=0)
    out_ref[:] = jnp.transpose(rows).reshape(G, K, 7)


def kernel(ctr_preds, ctr_feats, gt_boxes, points, gt_labels, box_w1, box_b1,
           box_gamma, box_beta, box_w2, box_b2, cls_w1, cls_b1, cls_gamma,
           cls_beta, cls_w2, cls_b2, mean_size):
    B, C, K = ctr_feats.shape
    G = B // _GRID

    full = pl.BlockSpec(index_map=lambda g: tuple([0]))
    full2 = pl.BlockSpec(index_map=lambda g: (0, 0))
    group3 = lambda d2, d3: pl.BlockSpec((G, d2, d3), lambda g: (g, 0, 0))

    return pl.pallas_call(
        _head_decode_kernel,
        grid=(_GRID,),
        in_specs=[
            group3(K, 3),
            group3(C, K),
            full2, full, full, full, full2, full,
            full2, full, full, full, full2, full,
            pl.BlockSpec(memory_space=pltpu.SMEM),
        ],
        out_specs=group3(K, 7),
        out_shape=jax.ShapeDtypeStruct((B, K, 7), jnp.float32),
    )(ctr_preds, ctr_feats,
      box_w1, box_b1, box_gamma, box_beta, box_w2, box_b2,
      cls_w1, cls_b1, cls_gamma, cls_beta, cls_w2, cls_b2,
      mean_size)
